# baseline (speedup 1.0000x reference)
import functools

import jax
import jax.numpy as jnp
from jax import lax
from jax.experimental import pallas as pl
from jax.experimental.pallas import tpu as pltpu
from jax.experimental.pallas import tpu_sc as plsc


@functools.cache
def _build_sc_gather(sc_rows, dim, rpw):
    active = sc_rows // rpw
    assert active * rpw == sc_rows and active <= 32 and rpw % 8 == 0
    mesh = plsc.VectorSubcoreMesh(core_axis_name="c", subcore_axis_name="s")

    @functools.partial(
        pl.kernel,
        mesh=mesh,
        out_type=jax.ShapeDtypeStruct((sc_rows, dim), jnp.float32),
        scratch_types=[
            pltpu.VMEM((rpw,), jnp.int32),
            pltpu.VMEM((rpw, dim), jnp.float32),
            pltpu.SemaphoreType.DMA,
        ],
    )
    def sc_gather(table_hbm, tok_hbm, out_hbm, idx_v, rows_v, sem):
        wid = lax.axis_index("s") * 2 + lax.axis_index("c")

        @pl.when(wid < active)
        def _():
            base = wid * rpw
            pltpu.sync_copy(tok_hbm.at[pl.ds(base, rpw)], idx_v)
            pltpu.async_copy(table_hbm.at[idx_v], rows_v, sem).wait()
            pltpu.sync_copy(rows_v, out_hbm.at[pl.ds(base, rpw)])

    return sc_gather


_RPB = 8


@functools.cache
def _build_tc_gather(tc_rows, dim):
    assert tc_rows % _RPB == 0
    grid = (tc_rows // _RPB,)

    def body(idx_ref, *refs):
        ins = refs[:_RPB]
        out_ref = refs[_RPB]
        for j in range(_RPB):
            out_ref[pl.ds(j, 1), :] = ins[j][0]

    in_specs = [
        pl.BlockSpec(
            (1, 1, dim), (lambda i, idx_ref, j=j: (idx_ref[_RPB * i + j], 0, 0))
        )
        for j in range(_RPB)
    ]
    return pl.pallas_call(
        body,
        grid_spec=pltpu.PrefetchScalarGridSpec(
            num_scalar_prefetch=1,
            grid=grid,
            in_specs=[in_specs[j] for j in range(_RPB)],
            out_specs=pl.BlockSpec((_RPB, dim), lambda i, idx_ref: (i, 0)),
        ),
        out_shape=jax.ShapeDtypeStruct((tc_rows, dim), jnp.float32),
    )


def kernel(sequence_embedding, tokens):
    batch, seq_len, dim = sequence_embedding.shape
    _, tokens_per_batch = tokens.shape
    rows = batch * tokens_per_batch
    table = sequence_embedding.reshape(batch * seq_len, dim)
    offsets = jnp.arange(batch, dtype=tokens.dtype)[:, None] * seq_len
    flat_tokens = (tokens + offsets).reshape(rows)

    sc_rows = 416
    sc_out = _build_sc_gather(sc_rows, dim, 32)(table, flat_tokens[:sc_rows])
    table3 = table[:, None, :]
    tc_out = _build_tc_gather(rows - sc_rows, dim)(
        flat_tokens[sc_rows:], *([table3] * _RPB)
    )
    out = jnp.concatenate([sc_out, tc_out], axis=0)
    return out.reshape(batch, tokens_per_batch * dim)


# speedup vs baseline: 9.4362x; 9.4362x over previous
import functools

import jax
import jax.numpy as jnp
from jax import lax
from jax.experimental import pallas as pl
from jax.experimental.pallas import tpu as pltpu
from jax.experimental.pallas import tpu_sc as plsc


@functools.cache
def _build_sc_gather(sc_rows, dim, rpw):
    active = sc_rows // rpw
    assert active * rpw == sc_rows and active <= 32 and rpw % 8 == 0
    mesh = plsc.VectorSubcoreMesh(core_axis_name="c", subcore_axis_name="s")

    @functools.partial(
        pl.kernel,
        mesh=mesh,
        out_type=jax.ShapeDtypeStruct((sc_rows, dim), jnp.float32),
        scratch_types=[
            pltpu.VMEM((rpw,), jnp.int32),
            pltpu.VMEM((rpw, dim), jnp.float32),
            pltpu.SemaphoreType.DMA,
        ],
    )
    def sc_gather(table_hbm, tok_hbm, out_hbm, idx_v, rows_v, sem):
        wid = lax.axis_index("s") * 2 + lax.axis_index("c")

        @pl.when(wid < active)
        def _():
            base = wid * rpw
            pltpu.sync_copy(tok_hbm.at[pl.ds(base, rpw)], idx_v)
            pltpu.async_copy(table_hbm.at[idx_v], rows_v, sem).wait()
            pltpu.sync_copy(rows_v, out_hbm.at[pl.ds(base, rpw)])

    return sc_gather


_RPB = 8


@functools.cache
def _build_tc_gather(tc_rows, dim):
    assert tc_rows % _RPB == 0
    grid = (tc_rows // _RPB,)

    def body(g_ref, r_ref, *refs):
        i = pl.program_id(0)
        ins = refs[:_RPB]
        out_ref = refs[_RPB]
        for j in range(_RPB):
            r = r_ref[_RPB * i + j]
            out_ref[pl.ds(j, 1), :] = ins[j][0, pl.ds(r, 1), :]

    in_specs = [
        pl.BlockSpec(
            (1, 8, dim),
            (lambda i, g_ref, r_ref, j=j: (g_ref[_RPB * i + j], 0, 0)),
        )
        for j in range(_RPB)
    ]
    return pl.pallas_call(
        body,
        grid_spec=pltpu.PrefetchScalarGridSpec(
            num_scalar_prefetch=2,
            grid=grid,
            in_specs=[in_specs[j] for j in range(_RPB)],
            out_specs=pl.BlockSpec((_RPB, dim), lambda i, g_ref, r_ref: (i, 0)),
        ),
        out_shape=jax.ShapeDtypeStruct((tc_rows, dim), jnp.float32),
    )


def kernel(sequence_embedding, tokens):
    batch, seq_len, dim = sequence_embedding.shape
    _, tokens_per_batch = tokens.shape
    rows = batch * tokens_per_batch
    table = sequence_embedding.reshape(batch * seq_len, dim)
    offsets = jnp.arange(batch, dtype=tokens.dtype)[:, None] * seq_len
    flat_tokens = (tokens + offsets).reshape(rows)

    sc_rows = 416
    sc_out = _build_sc_gather(sc_rows, dim, 32)(table, flat_tokens[:sc_rows])

    tc_idx = flat_tokens[sc_rows:]
    groups = tc_idx // 8
    rowsel = tc_idx % 8
    table_g = sequence_embedding.reshape(batch * seq_len // 8, 8, dim)
    tc_out = _build_tc_gather(rows - sc_rows, dim)(
        groups, rowsel, *([table_g] * _RPB)
    )
    out = jnp.concatenate([sc_out, tc_out], axis=0)
    return out.reshape(batch, tokens_per_batch * dim)


# speedup vs baseline: 17.1139x; 1.8136x over previous
import functools

import jax
import jax.numpy as jnp
from jax import lax
from jax.experimental import pallas as pl
from jax.experimental.pallas import tpu as pltpu
from jax.experimental.pallas import tpu_sc as plsc

_LANES = 16


@functools.cache
def _build_gather(rows, seq_len, dim, tokens_per_batch, rpw, num_workers):
    active = rows // rpw
    assert active * rpw == rows and active <= num_workers
    mesh = plsc.VectorSubcoreMesh(core_axis_name="c", subcore_axis_name="s")

    nch = 2
    cpw = rpw // nch
    assert cpw % 8 == 0

    @functools.partial(
        pl.kernel,
        mesh=mesh,
        out_type=jax.ShapeDtypeStruct((rows, dim), jnp.float32),
        scratch_types=[
            pltpu.VMEM((rpw,), jnp.int32),
            pltpu.VMEM((rpw, dim), jnp.float32),
            pltpu.SemaphoreType.DMA,
            pltpu.SemaphoreType.DMA,
            pltpu.SemaphoreType.DMA,
        ],
    )
    def gather_kernel(table_hbm, tok_hbm, out_hbm, idx_v, rows_v, gsem0, gsem1, wsem):
        wid = lax.axis_index("s") * 2 + lax.axis_index("c")

        @pl.when(wid < active)
        def _():
            base = wid * rpw
            pltpu.sync_copy(tok_hbm.at[pl.ds(base, rpw)], idx_v)
            gsems = [gsem0, gsem1]
            gathers = [
                pltpu.async_copy(
                    table_hbm.at[idx_v.at[pl.ds(ch * cpw, cpw)]],
                    rows_v.at[pl.ds(ch * cpw, cpw)],
                    gsems[ch],
                )
                for ch in range(nch)
            ]
            writes = []
            for ch in range(nch):
                gathers[ch].wait()
                writes.append(
                    pltpu.async_copy(
                        rows_v.at[pl.ds(ch * cpw, cpw)],
                        out_hbm.at[pl.ds(base + ch * cpw, cpw)],
                        wsem,
                    )
                )
            for w in writes:
                w.wait()

    return gather_kernel


def kernel(sequence_embedding, tokens):
    batch, seq_len, dim = sequence_embedding.shape
    _, tokens_per_batch = tokens.shape
    rows = batch * tokens_per_batch
    table = sequence_embedding.reshape(batch * seq_len, dim)
    offsets = jnp.arange(batch, dtype=tokens.dtype)[:, None] * seq_len
    flat_tokens = (tokens + offsets).reshape(rows)
    gather = _build_gather(rows, seq_len, dim, tokens_per_batch, 32, 32)
    out = gather(table, flat_tokens)
    return out.reshape(batch, tokens_per_batch * dim)
